# baseline (speedup 1.0000x reference)
import functools

import jax
import jax.numpy as jnp
from jax import lax
from jax.experimental import pallas as pl
from jax.experimental.pallas import tpu as pltpu
from jax.experimental.pallas import tpu_sc as plsc

MAX_DEG = 6
N_PER_DEG = 14000
N_ATOMS = (MAX_DEG + 1) * N_PER_DEG
FEAT = 128
BATCH = 64
NPASS = MAX_DEG * (MAX_DEG + 1) // 2

NC = 2
NS = 16
NW = NC * NS
PAD_N = 14080
CHUNK = PAD_N // NW
SUB = 88
NSUB = CHUNK // SUB

BLK = 1000
NBLK = N_PER_DEG // BLK

_PASS_DS = [(d, s) for d in range(1, MAX_DEG + 1) for s in range(d)]


def _sc_gather_body(atoms_hbm, idx_hbm, g1, g2, g3, g4, g5, g6,
                    idx_f, rows_v, sem_g, sem_s):
    wid = lax.axis_index("s") * NC + lax.axis_index("c")
    base = wid * CHUNK
    outs = [g1, g2, g3, g4, g5, g6]

    pltpu.sync_copy(idx_hbm.at[pl.ds(wid * (NPASS * CHUNK), NPASS * CHUNK)],
                    idx_f)

    def fire(p):
        ph = p % 2
        return [pltpu.async_copy(
            atoms_hbm.at[idx_f.at[pl.ds(p * CHUNK + c * SUB, SUB)]],
            rows_v.at[ph, pl.ds(c * SUB, SUB)], sem_g)
            for c in range(NSUB)]

    def store(p):
        d, s = _PASS_DS[p]
        return pltpu.async_copy(rows_v.at[p % 2],
                                outs[d - 1].at[s, pl.ds(base, CHUNK)], sem_s)

    gh = fire(0)
    sh = None
    for p in range(NPASS):
        for h in gh:
            h.wait()
        if sh is not None:
            sh.wait()
        if p + 1 < NPASS:
            gh = fire(p + 1)
        sh = store(p)
    sh.wait()


@functools.cache
def _make_sc_gather():
    return pl.kernel(
        _sc_gather_body,
        out_type=[jax.ShapeDtypeStruct((d, PAD_N, FEAT), jnp.float32)
                  for d in range(1, MAX_DEG + 1)],
        mesh=plsc.VectorSubcoreMesh(core_axis_name="c", subcore_axis_name="s",
                                    num_cores=NC, num_subcores=NS),
        scratch_types=[
            pltpu.VMEM((NPASS * CHUNK,), jnp.int32),
            pltpu.VMEM((2, CHUNK, FEAT), jnp.float32),
            pltpu.SemaphoreType.DMA,
            pltpu.SemaphoreType.DMA,
        ],
    )


def _sc_gather(atoms, idx_tiles):
    return _make_sc_gather()(atoms, idx_tiles)


def _tc_body(atoms_ref, g1, g2, g3, g4, g5, g6, wself, wrel, wgath,
             bact, bgath, mem_ref, act_out, gath_out):
    d = pl.program_id(0)
    j = pl.program_id(1)
    a = atoms_ref[...]

    gs = [g1, g2, g3, g4, g5, g6]
    ns = lax.switch(
        d,
        [lambda: jnp.zeros_like(a)]
        + [(lambda dd=dd: jnp.sum(gs[dd - 1][...], axis=0))
           for dd in range(1, MAX_DEG + 1)])

    act = (jnp.dot(ns, wrel[0], preferred_element_type=jnp.float32)
           + jnp.dot(a, wself[0], preferred_element_type=jnp.float32)
           + bact[0])
    act_out[...] = act

    g = jnp.dot(a, wgath[0], preferred_element_type=jnp.float32) + bgath[0]
    m = mem_ref[0, 0]
    onehot = (lax.broadcasted_iota(jnp.int32, (BATCH, BLK), 0)
              == m[None, :]).astype(jnp.float32)
    part = jnp.dot(onehot, g, preferred_element_type=jnp.float32)

    first = (d == 0) & (j == 0)

    @pl.when(first)
    def _():
        gath_out[...] = part

    @pl.when(jnp.logical_not(first))
    def _():
        gath_out[...] += part


def _tc_affine(atoms, gbufs, wself, wrel, wgath, bact, bgath, mem_r):
    g_specs = [
        pl.BlockSpec((dd, BLK, FEAT),
                     lambda d, j, dd=dd: (0, jnp.where(d == dd, j, 0), 0))
        for dd in range(1, MAX_DEG + 1)
    ]
    return pl.pallas_call(
        _tc_body,
        grid=(MAX_DEG + 1, NBLK),
        in_specs=[
            pl.BlockSpec((BLK, FEAT), lambda d, j: (d * NBLK + j, 0)),
            *g_specs,
            pl.BlockSpec((1, FEAT, FEAT), lambda d, j: (d, 0, 0)),
            pl.BlockSpec((1, FEAT, FEAT), lambda d, j: (d, 0, 0)),
            pl.BlockSpec((1, FEAT, FEAT), lambda d, j: (d, 0, 0)),
            pl.BlockSpec((1, 1, FEAT), lambda d, j: (d, 0, 0)),
            pl.BlockSpec((1, 1, FEAT), lambda d, j: (d, 0, 0)),
            pl.BlockSpec((1, 1, BLK), lambda d, j: (d * NBLK + j, 0, 0)),
        ],
        out_specs=[
            pl.BlockSpec((BLK, FEAT), lambda d, j: (d * NBLK + j, 0)),
            pl.BlockSpec((BATCH, FEAT), lambda d, j: (0, 0)),
        ],
        out_shape=[
            jax.ShapeDtypeStruct((N_ATOMS, FEAT), jnp.float32),
            jax.ShapeDtypeStruct((BATCH, FEAT), jnp.float32),
        ],
        compiler_params=pltpu.CompilerParams(
            dimension_semantics=("arbitrary", "arbitrary")),
    )(atoms, *gbufs, wself, wrel, wgath, bact, bgath, mem_r)


def kernel(atoms, deg_slice, membership, deg_adj_1, deg_adj_2, deg_adj_3,
           deg_adj_4, deg_adj_5, deg_adj_6, W_stack, b_stack):
    adjs = [deg_adj_1, deg_adj_2, deg_adj_3, deg_adj_4, deg_adj_5, deg_adj_6]
    idx_rows = jnp.concatenate([a.T for a in adjs], axis=0)
    idx_rows = jnp.pad(idx_rows, ((0, 0), (0, PAD_N - N_PER_DEG)))
    idx_tiles = idx_rows.reshape(NPASS, NW, CHUNK).transpose(1, 0, 2).reshape(-1)

    gbufs = _sc_gather(atoms, idx_tiles)

    i_self = jnp.array([12, 1, 3, 5, 7, 9, 11], dtype=jnp.int32)
    i_gath = jnp.array([19, 13, 14, 15, 16, 17, 18], dtype=jnp.int32)
    i_rel = jnp.array([0, 0, 2, 4, 6, 8, 10], dtype=jnp.int32)
    wself = W_stack[i_self]
    wgath = W_stack[i_gath]
    wrel = W_stack[i_rel].at[0].set(0.0)
    bact = (b_stack[i_self] + b_stack[i_rel].at[0].set(0.0)).reshape(
        MAX_DEG + 1, 1, FEAT)
    bgath = b_stack[i_gath].reshape(MAX_DEG + 1, 1, FEAT)
    mem_r = membership.reshape(N_ATOMS // BLK, 1, BLK)

    activated, atom_gather = _tc_affine(
        atoms, gbufs, wself, wrel, wgath, bact, bgath, mem_r)
    return activated, atom_gather


# speedup vs baseline: 1.1995x; 1.1995x over previous
import functools

import jax
import jax.numpy as jnp
from jax import lax
from jax.experimental import pallas as pl
from jax.experimental.pallas import tpu as pltpu
from jax.experimental.pallas import tpu_sc as plsc

MAX_DEG = 6
N_PER_DEG = 14000
N_ATOMS = (MAX_DEG + 1) * N_PER_DEG
FEAT = 128
BATCH = 64
NPASS = MAX_DEG * (MAX_DEG + 1) // 2

NC = 2
NS = 16
NW = NC * NS
PAD_N = 14080
CHUNK = PAD_N // NW
SUB = 88
NSUB = CHUNK // SUB

BLK = 1000
NBLK = N_PER_DEG // BLK

_PASS_DS = [(d, s) for d in range(1, MAX_DEG + 1) for s in range(d)]
P_SPLIT = 10


def _sc_gather_body(p_lo, p_hi, atoms_hbm, idx_hbm, *rest):
    n_out = len(set(d for d, _ in _PASS_DS[p_lo:p_hi]))
    outs = rest[:n_out]
    idx_f, rows_v, sem_g, sem_s = rest[n_out:]
    d_lo = _PASS_DS[p_lo][0]

    wid = lax.axis_index("s") * NC + lax.axis_index("c")
    base = wid * CHUNK
    npass = p_hi - p_lo

    pltpu.sync_copy(
        idx_hbm.at[pl.ds(wid * (NPASS * CHUNK) + p_lo * CHUNK, npass * CHUNK)],
        idx_f.at[pl.ds(0, npass * CHUNK)])

    def fire(p):
        ph = p % 2
        return [pltpu.async_copy(
            atoms_hbm.at[idx_f.at[pl.ds(p * CHUNK + c * SUB, SUB)]],
            rows_v.at[ph, pl.ds(c * SUB, SUB)], sem_g)
            for c in range(NSUB)]

    def store(p):
        d, s = _PASS_DS[p_lo + p]
        return pltpu.async_copy(rows_v.at[p % 2],
                                outs[d - d_lo].at[s, pl.ds(base, CHUNK)],
                                sem_s)

    gh = fire(0)
    sh = None
    for p in range(npass):
        for h in gh:
            h.wait()
        if sh is not None:
            sh.wait()
        if p + 1 < npass:
            gh = fire(p + 1)
        sh = store(p)
    sh.wait()


@functools.cache
def _make_sc_gather(p_lo, p_hi):
    degs = sorted(set(d for d, _ in _PASS_DS[p_lo:p_hi]))
    npass = p_hi - p_lo
    return pl.kernel(
        functools.partial(_sc_gather_body, p_lo, p_hi),
        out_type=[jax.ShapeDtypeStruct((d, PAD_N, FEAT), jnp.float32)
                  for d in degs],
        mesh=plsc.VectorSubcoreMesh(core_axis_name="c", subcore_axis_name="s",
                                    num_cores=NC, num_subcores=NS),
        scratch_types=[
            pltpu.VMEM((npass * CHUNK,), jnp.int32),
            pltpu.VMEM((2, CHUNK, FEAT), jnp.float32),
            pltpu.SemaphoreType.DMA,
            pltpu.SemaphoreType.DMA,
        ],
    )


def _tc_body_lo(atoms_ref, g1, g2, g3, g4, wself, wrel, wgath,
                bact, bgath, mem_ref, act_out, gath_out):
    d = pl.program_id(0)
    j = pl.program_id(1)
    a = atoms_ref[...]

    gs = [g1, g2, g3, g4]
    ns = jnp.zeros_like(a)
    for dd in range(1, 5):
        ns = jnp.where(d == dd, jnp.sum(gs[dd - 1][...], axis=0), ns)

    act = (jnp.dot(ns, wrel[0], preferred_element_type=jnp.float32)
           + jnp.dot(a, wself[0], preferred_element_type=jnp.float32)
           + bact[0])
    act_out[...] = act

    g = jnp.dot(a, wgath[0], preferred_element_type=jnp.float32) + bgath[0]
    m = mem_ref[0, 0]
    onehot = (lax.broadcasted_iota(jnp.int32, (BATCH, BLK), 0)
              == m[None, :]).astype(jnp.float32)
    part = jnp.dot(onehot, g, preferred_element_type=jnp.float32)

    first = (d == 0) & (j == 0)

    @pl.when(first)
    def _():
        gath_out[...] = part

    @pl.when(jnp.logical_not(first))
    def _():
        gath_out[...] += part


def _tc_body_hi(atoms_ref, g5, g6, acc_ref, act_in, wself, wrel, wgath,
                bact, bgath, mem_ref, act_out, gath_out):
    del act_in
    d = pl.program_id(0)
    j = pl.program_id(1)
    a = atoms_ref[...]

    s5 = jnp.sum(g5[...], axis=0)
    s6 = jnp.sum(g6[...], axis=0)
    ns = jnp.where(d == 0, s5, s6)

    act = (jnp.dot(ns, wrel[0], preferred_element_type=jnp.float32)
           + jnp.dot(a, wself[0], preferred_element_type=jnp.float32)
           + bact[0])
    act_out[...] = act

    g = jnp.dot(a, wgath[0], preferred_element_type=jnp.float32) + bgath[0]
    m = mem_ref[0, 0]
    onehot = (lax.broadcasted_iota(jnp.int32, (BATCH, BLK), 0)
              == m[None, :]).astype(jnp.float32)
    part = jnp.dot(onehot, g, preferred_element_type=jnp.float32)

    first = (d == 0) & (j == 0)

    @pl.when(first)
    def _():
        gath_out[...] = acc_ref[...] + part

    @pl.when(jnp.logical_not(first))
    def _():
        gath_out[...] += part


def _tc_affine_lo(atoms, gbufs, wself, wrel, wgath, bact, bgath, mem_r):
    g_specs = [
        pl.BlockSpec((dd, BLK, FEAT),
                     lambda d, j, dd=dd: (0, jnp.where(d == dd, j, 0), 0))
        for dd in range(1, 5)
    ]
    return pl.pallas_call(
        _tc_body_lo,
        grid=(5, NBLK),
        in_specs=[
            pl.BlockSpec((BLK, FEAT), lambda d, j: (d * NBLK + j, 0)),
            *g_specs,
            pl.BlockSpec((1, FEAT, FEAT), lambda d, j: (d, 0, 0)),
            pl.BlockSpec((1, FEAT, FEAT), lambda d, j: (d, 0, 0)),
            pl.BlockSpec((1, FEAT, FEAT), lambda d, j: (d, 0, 0)),
            pl.BlockSpec((1, 1, FEAT), lambda d, j: (d, 0, 0)),
            pl.BlockSpec((1, 1, FEAT), lambda d, j: (d, 0, 0)),
            pl.BlockSpec((1, 1, BLK), lambda d, j: (d * NBLK + j, 0, 0)),
        ],
        out_specs=[
            pl.BlockSpec((BLK, FEAT), lambda d, j: (d * NBLK + j, 0)),
            pl.BlockSpec((BATCH, FEAT), lambda d, j: (0, 0)),
        ],
        out_shape=[
            jax.ShapeDtypeStruct((N_ATOMS, FEAT), jnp.float32),
            jax.ShapeDtypeStruct((BATCH, FEAT), jnp.float32),
        ],
        compiler_params=pltpu.CompilerParams(
            dimension_semantics=("arbitrary", "arbitrary")),
    )(atoms, *gbufs, wself, wrel, wgath, bact, bgath, mem_r)


def _tc_affine_hi(atoms, g5, g6, acc, act_part,
                  wself, wrel, wgath, bact, bgath, mem_r):
    return pl.pallas_call(
        _tc_body_hi,
        grid=(2, NBLK),
        in_specs=[
            pl.BlockSpec((BLK, FEAT), lambda d, j: ((d + 5) * NBLK + j, 0)),
            pl.BlockSpec((5, BLK, FEAT),
                         lambda d, j: (0, jnp.where(d == 0, j, 0), 0)),
            pl.BlockSpec((6, BLK, FEAT),
                         lambda d, j: (0, jnp.where(d == 1, j, 0), 0)),
            pl.BlockSpec((BATCH, FEAT), lambda d, j: (0, 0)),
            pl.BlockSpec(memory_space=pl.ANY),
            pl.BlockSpec((1, FEAT, FEAT), lambda d, j: (d, 0, 0)),
            pl.BlockSpec((1, FEAT, FEAT), lambda d, j: (d, 0, 0)),
            pl.BlockSpec((1, FEAT, FEAT), lambda d, j: (d, 0, 0)),
            pl.BlockSpec((1, 1, FEAT), lambda d, j: (d, 0, 0)),
            pl.BlockSpec((1, 1, FEAT), lambda d, j: (d, 0, 0)),
            pl.BlockSpec((1, 1, BLK),
                         lambda d, j: ((d + 5) * NBLK + j, 0, 0)),
        ],
        out_specs=[
            pl.BlockSpec((BLK, FEAT), lambda d, j: ((d + 5) * NBLK + j, 0)),
            pl.BlockSpec((BATCH, FEAT), lambda d, j: (0, 0)),
        ],
        out_shape=[
            jax.ShapeDtypeStruct((N_ATOMS, FEAT), jnp.float32),
            jax.ShapeDtypeStruct((BATCH, FEAT), jnp.float32),
        ],
        input_output_aliases={4: 0},
        compiler_params=pltpu.CompilerParams(
            dimension_semantics=("arbitrary", "arbitrary")),
    )(atoms, g5, g6, acc, act_part, wself, wrel, wgath, bact, bgath, mem_r)


def kernel(atoms, deg_slice, membership, deg_adj_1, deg_adj_2, deg_adj_3,
           deg_adj_4, deg_adj_5, deg_adj_6, W_stack, b_stack):
    adjs = [deg_adj_1, deg_adj_2, deg_adj_3, deg_adj_4, deg_adj_5, deg_adj_6]
    idx_rows = jnp.concatenate([a.T for a in adjs], axis=0)
    idx_rows = jnp.pad(idx_rows, ((0, 0), (0, PAD_N - N_PER_DEG)))
    idx_tiles = idx_rows.reshape(NPASS, NW, CHUNK).transpose(1, 0, 2).reshape(-1)

    g1, g2, g3, g4 = _make_sc_gather(0, P_SPLIT)(atoms, idx_tiles)
    g5, g6 = _make_sc_gather(P_SPLIT, NPASS)(atoms, idx_tiles)

    i_self = jnp.array([12, 1, 3, 5, 7, 9, 11], dtype=jnp.int32)
    i_gath = jnp.array([19, 13, 14, 15, 16, 17, 18], dtype=jnp.int32)
    i_rel = jnp.array([0, 0, 2, 4, 6, 8, 10], dtype=jnp.int32)
    wself = W_stack[i_self]
    wgath = W_stack[i_gath]
    wrel = W_stack[i_rel].at[0].set(0.0)
    bact = (b_stack[i_self] + b_stack[i_rel].at[0].set(0.0)).reshape(
        MAX_DEG + 1, 1, FEAT)
    bgath = b_stack[i_gath].reshape(MAX_DEG + 1, 1, FEAT)
    mem_r = membership.reshape(N_ATOMS // BLK, 1, BLK)

    act_part, acc = _tc_affine_lo(
        atoms, [g1, g2, g3, g4], wself[:5], wrel[:5], wgath[:5],
        bact[:5], bgath[:5], mem_r)
    activated, atom_gather = _tc_affine_hi(
        atoms, g5, g6, acc, act_part, wself[5:], wrel[5:], wgath[5:],
        bact[5:], bgath[5:], mem_r)
    return activated, atom_gather
